# baseline (speedup 1.0000x reference)
import functools
import math

import jax
import jax.numpy as jnp
from jax import lax
from jax.experimental import pallas as pl
from jax.experimental.pallas import tpu as pltpu
from jax.experimental.pallas import tpu_sc as plsc

_VOCAB = 100000
_D = 2048
_BATCH = 4
_SEQ = 2048
_B = _BATCH * _SEQ
_SCALE = math.sqrt(_D)

_NC = 2
_NS = 16
_NW = _NC * _NS
_BPW = _B // _NW
_LANES = 16
_CHUNK = 8
_NCH = _BPW // _CHUNK
_NT = _NCH // 2
_VPR = _D // _LANES

_mesh = plsc.VectorSubcoreMesh(core_axis_name="c", subcore_axis_name="s")

_GDN = lax.GatherDimensionNumbers(
    offset_dims=(), collapsed_slice_dims=(0,), start_index_map=(0,)
)


def _splat(vec, lane):
    return lax.gather(
        vec,
        jnp.full((_LANES, 1), lane, jnp.int32),
        _GDN,
        slice_sizes=(1,),
        mode=lax.GatherScatterMode.PROMISE_IN_BOUNDS,
    )


@functools.partial(
    pl.kernel,
    mesh=_mesh,
    out_type=jax.ShapeDtypeStruct((_BATCH, _SEQ, 1, _D), jnp.float32),
    scratch_types=[
        pltpu.VMEM((_BPW,), jnp.int32),
        pltpu.VMEM((_CHUNK, _D), jnp.float32),
        pltpu.VMEM((_CHUNK, _D), jnp.float32),
        pltpu.VMEM((_CHUNK, 1, _D), jnp.float32),
        pltpu.VMEM((_CHUNK, 1, _D), jnp.float32),
        pltpu.SemaphoreType.DMA,
        pltpu.SemaphoreType.DMA,
        pltpu.SemaphoreType.DMA,
        pltpu.SemaphoreType.DMA,
    ],
)
def _emb_lookup(
    idx_hbm, table_hbm, out_hbm,
    idx_v, g0, g1, s0, s1, gsem0, gsem1, ssem0, ssem1,
):
    wid = lax.axis_index("s") * _NC + lax.axis_index("c")
    base = wid * _BPW
    bi = base // _SEQ
    sbase = base % _SEQ
    pltpu.sync_copy(idx_hbm.at[pl.ds(base, _BPW)], idx_v)

    gbuf = (g0, g1)
    sbuf = (s0, s1)
    gsem = (gsem0, gsem1)
    ssem = (ssem0, ssem1)

    def issue_gather(k, b):
        pltpu.async_copy(
            table_hbm.at[idx_v.at[pl.ds(k * _CHUNK, _CHUNK)]], gbuf[b], gsem[b]
        )

    def wait_gather(b):
        pltpu.make_async_copy(
            table_hbm.at[idx_v.at[pl.ds(0, _CHUNK)]], gbuf[b], gsem[b]
        ).wait()

    def issue_store(k, b):
        pltpu.async_copy(
            sbuf[b], out_hbm.at[bi, pl.ds(sbase + k * _CHUNK, _CHUNK)], ssem[b]
        )

    def wait_store(b):
        pltpu.make_async_copy(
            sbuf[b], out_hbm.at[bi, pl.ds(sbase, _CHUNK)], ssem[b]
        ).wait()

    def process(t, b, first=False, last=False):
        k = 2 * t + b
        wait_gather(b)
        if not first:
            wait_store(b)
        iv = idx_v[pl.ds(t * _LANES, _LANES)]
        sv = jnp.where(iv != 0, jnp.float32(_SCALE), jnp.float32(0.0))
        splats = [_splat(sv, b * _CHUNK + rr) for rr in range(_CHUNK)]

        def jbody(j, c):
            sl = pl.ds(j * _LANES, _LANES)
            for rr in range(_CHUNK):
                sbuf[b][rr, 0, sl] = gbuf[b][rr, sl] * splats[rr]
            return c

        lax.fori_loop(0, _VPR, jbody, 0)
        if not last:
            issue_gather(k + 2, b)
        issue_store(k, b)

    issue_gather(0, 0)
    issue_gather(1, 1)
    process(0, 0, first=True)
    process(0, 1, first=True)

    def tbody(t, c):
        process(t, 0)
        process(t, 1)
        return c

    lax.fori_loop(1, _NT - 1, tbody, 0)

    process(_NT - 1, 0, last=True)
    process(_NT - 1, 1, last=True)
    wait_store(0)
    wait_store(1)


def kernel(x, embedding_weights):
    idx = x.reshape(-1).astype(jnp.int32)
    return _emb_lookup(idx, embedding_weights)


# speedup vs baseline: 1.3612x; 1.3612x over previous
import functools
import math

import jax
import jax.numpy as jnp
from jax import lax
from jax.experimental import pallas as pl
from jax.experimental.pallas import tpu as pltpu
from jax.experimental.pallas import tpu_sc as plsc

_VOCAB = 100000
_D = 2048
_BATCH = 4
_SEQ = 2048
_B = _BATCH * _SEQ
_SCALE = math.sqrt(_D)

_NC = 2
_NS = 16
_NW = _NC * _NS
_BPW = _B // _NW
_LANES = 16
_CHUNK = 8
_NCH = _BPW // _CHUNK
_NT = _NCH // 2
_VPR = _D // _LANES

_mesh = plsc.VectorSubcoreMesh(core_axis_name="c", subcore_axis_name="s")

_GDN = lax.GatherDimensionNumbers(
    offset_dims=(), collapsed_slice_dims=(0,), start_index_map=(0,)
)


def _splat(vec, lane):
    return lax.gather(
        vec,
        jnp.full((_LANES, 1), lane, jnp.int32),
        _GDN,
        slice_sizes=(1,),
        mode=lax.GatherScatterMode.PROMISE_IN_BOUNDS,
    )


@functools.partial(
    pl.kernel,
    mesh=_mesh,
    out_type=jax.ShapeDtypeStruct((_BATCH, _SEQ, _D), jnp.float32),
    scratch_types=[
        pltpu.VMEM((_BPW,), jnp.int32),
        pltpu.VMEM((_CHUNK, _D), jnp.float32),
        pltpu.VMEM((_CHUNK, _D), jnp.float32),
        pltpu.VMEM((_CHUNK, _D), jnp.float32),
        pltpu.VMEM((_CHUNK, _D), jnp.float32),
        pltpu.SemaphoreType.DMA,
        pltpu.SemaphoreType.DMA,
        pltpu.SemaphoreType.DMA,
        pltpu.SemaphoreType.DMA,
    ],
)
def _emb_lookup(
    idx_hbm, table_hbm, out_hbm,
    idx_v, g0, g1, s0, s1, gsem0, gsem1, ssem0, ssem1,
):
    wid = lax.axis_index("s") * _NC + lax.axis_index("c")
    base = wid * _BPW
    bi = base // _SEQ
    sbase = base % _SEQ
    pltpu.sync_copy(idx_hbm.at[pl.ds(base, _BPW)], idx_v)

    gbuf = (g0, g1)
    sbuf = (s0, s1)
    gsem = (gsem0, gsem1)
    ssem = (ssem0, ssem1)

    def issue_gather(k, b):
        pltpu.async_copy(
            table_hbm.at[idx_v.at[pl.ds(k * _CHUNK, _CHUNK)]], gbuf[b], gsem[b]
        )

    def wait_gather(b):
        pltpu.make_async_copy(
            table_hbm.at[idx_v.at[pl.ds(0, _CHUNK)]], gbuf[b], gsem[b]
        ).wait()

    def issue_store(k, b):
        pltpu.async_copy(
            sbuf[b], out_hbm.at[bi, pl.ds(sbase + k * _CHUNK, _CHUNK)], ssem[b]
        )

    def wait_store(b):
        pltpu.make_async_copy(
            sbuf[b], out_hbm.at[bi, pl.ds(sbase, _CHUNK)], ssem[b]
        ).wait()

    def process(t, b, first=False, last=False):
        k = 2 * t + b
        wait_gather(b)
        if not first:
            wait_store(b)
        iv = idx_v[pl.ds(t * _LANES, _LANES)]
        sv = jnp.where(iv != 0, jnp.float32(_SCALE), jnp.float32(0.0))
        splats = [_splat(sv, b * _CHUNK + rr) for rr in range(_CHUNK)]

        def jbody(j, c):
            sl = pl.ds(j * _LANES, _LANES)
            for rr in range(_CHUNK):
                sbuf[b][rr, sl] = gbuf[b][rr, sl] * splats[rr]
            return c

        lax.fori_loop(0, _VPR, jbody, 0)
        if not last:
            issue_gather(k + 2, b)
        issue_store(k, b)

    issue_gather(0, 0)
    issue_gather(1, 1)
    process(0, 0, first=True)
    process(0, 1, first=True)

    def tbody(t, c):
        process(t, 0)
        process(t, 1)
        return c

    lax.fori_loop(1, _NT - 1, tbody, 0)

    process(_NT - 1, 0, last=True)
    process(_NT - 1, 1, last=True)
    wait_store(0)
    wait_store(1)


def kernel(x, embedding_weights):
    idx = x.reshape(-1).astype(jnp.int32)
    out = _emb_lookup(idx, embedding_weights)
    return jnp.expand_dims(out, 2)


# speedup vs baseline: 1.3620x; 1.0005x over previous
import functools
import math

import jax
import jax.numpy as jnp
from jax import lax
from jax.experimental import pallas as pl
from jax.experimental.pallas import tpu as pltpu
from jax.experimental.pallas import tpu_sc as plsc

_VOCAB = 100000
_D = 2048
_BATCH = 4
_SEQ = 2048
_B = _BATCH * _SEQ
_SCALE = math.sqrt(_D)

_NC = 2
_NS = 16
_NW = _NC * _NS
_BPW = _B // _NW
_LANES = 16
_CHUNK = 8
_NCH = _BPW // _CHUNK
_NT = _NCH // 2
_VPR = _D // _LANES

_mesh = plsc.VectorSubcoreMesh(core_axis_name="c", subcore_axis_name="s")

_GDN = lax.GatherDimensionNumbers(
    offset_dims=(), collapsed_slice_dims=(0,), start_index_map=(0,)
)


def _splat(vec, lane):
    return lax.gather(
        vec,
        jnp.full((_LANES, 1), lane, jnp.int32),
        _GDN,
        slice_sizes=(1,),
        mode=lax.GatherScatterMode.PROMISE_IN_BOUNDS,
    )


@functools.partial(
    pl.kernel,
    mesh=_mesh,
    out_type=jax.ShapeDtypeStruct((_BATCH, _SEQ, _D), jnp.float32),
    scratch_types=[
        pltpu.VMEM((_BPW,), jnp.int32),
        pltpu.VMEM((_CHUNK, _D), jnp.float32),
        pltpu.VMEM((_CHUNK, _D), jnp.float32),
        pltpu.VMEM((_CHUNK, _D), jnp.float32),
        pltpu.VMEM((_CHUNK, _D), jnp.float32),
        pltpu.SemaphoreType.DMA,
        pltpu.SemaphoreType.DMA,
        pltpu.SemaphoreType.DMA,
        pltpu.SemaphoreType.DMA,
    ],
)
def _emb_lookup(
    idx_hbm, table_hbm, out_hbm,
    idx_v, g0, g1, s0, s1, gsem0, gsem1, ssem0, ssem1,
):
    wid = lax.axis_index("s") * _NC + lax.axis_index("c")
    base = wid * _BPW
    bi = base // _SEQ
    sbase = base % _SEQ
    pltpu.sync_copy(idx_hbm.at[bi, pl.ds(sbase, _BPW)], idx_v)

    gbuf = (g0, g1)
    sbuf = (s0, s1)
    gsem = (gsem0, gsem1)
    ssem = (ssem0, ssem1)

    def issue_gather(k, b):
        pltpu.async_copy(
            table_hbm.at[idx_v.at[pl.ds(k * _CHUNK, _CHUNK)]], gbuf[b], gsem[b]
        )

    def wait_gather(b):
        pltpu.make_async_copy(
            table_hbm.at[idx_v.at[pl.ds(0, _CHUNK)]], gbuf[b], gsem[b]
        ).wait()

    def issue_store(k, b):
        pltpu.async_copy(
            sbuf[b], out_hbm.at[bi, pl.ds(sbase + k * _CHUNK, _CHUNK)], ssem[b]
        )

    def wait_store(b):
        pltpu.make_async_copy(
            sbuf[b], out_hbm.at[bi, pl.ds(sbase, _CHUNK)], ssem[b]
        ).wait()

    def process(t, b, first=False, last=False):
        k = 2 * t + b
        wait_gather(b)
        if not first:
            wait_store(b)
        iv = idx_v[pl.ds(t * _LANES, _LANES)]
        sv = jnp.where(iv != 0, jnp.float32(_SCALE), jnp.float32(0.0))
        splats = [_splat(sv, b * _CHUNK + rr) for rr in range(_CHUNK)]

        def jbody(j, c):
            sl = pl.ds(j * _LANES, _LANES)
            for rr in range(_CHUNK):
                sbuf[b][rr, sl] = gbuf[b][rr, sl] * splats[rr]
            return c

        lax.fori_loop(0, _VPR, jbody, 0)
        if not last:
            issue_gather(k + 2, b)
        issue_store(k, b)

    issue_gather(0, 0)
    issue_gather(1, 1)
    process(0, 0, first=True)
    process(0, 1, first=True)

    def tbody(t, c):
        process(t, 0)
        process(t, 1)
        return c

    lax.fori_loop(1, _NT - 1, tbody, 0)

    process(_NT - 1, 0, last=True)
    process(_NT - 1, 1, last=True)
    wait_store(0)
    wait_store(1)


def kernel(x, embedding_weights):
    out = _emb_lookup(x.astype(jnp.int32), embedding_weights)
    return jnp.expand_dims(out, 2)
